# Initial kernel scaffold; baseline (speedup 1.0000x reference)
import jax
import jax.numpy as jnp
from jax.experimental import pallas as pl


def kernel(x, edge_index, W1, W2):
    raise NotImplementedError("write your pallas kernel here")


# speedup vs baseline: 34.5733x; 34.5733x over previous
import functools

import jax
import jax.numpy as jnp
from jax import lax
from jax.experimental import pallas as pl
from jax.experimental.pallas import tpu as pltpu
from jax.experimental.pallas import tpu_sc as plsc

N_NODES = 10000
N_EDGES = 320000
D = 128

NC = 2
NS = 16
NW = NC * NS

N_PAD = 10240
E_PAD = 327680
EPT = E_PAD // NW
EB = 128
NB = EPT // EB
ROWS_PT = N_PAD // NS
RB = ROWS_PT // EB

_mesh = plsc.VectorSubcoreMesh(core_axis_name="c", subcore_axis_name="s")


HW = 16


@functools.partial(
    pl.kernel,
    out_type=jax.ShapeDtypeStruct((NC, N_PAD, HW), jnp.float32),
    mesh=_mesh,
    scratch_types=[
        pltpu.VMEM((NB, EB), jnp.int32),
        pltpu.VMEM((EB, HW), jnp.float32),
        pltpu.VMEM((EB, HW), jnp.float32),
        pltpu.VMEM_SHARED((N_PAD, HW), jnp.float32),
    ],
)
def _deg_kernel(dst_hbm, out_hbm, dst_v, msg_v, tmp_v, hist_shared):
    c = lax.axis_index("c")
    s = lax.axis_index("s")
    wid = s * NC + c
    pltpu.sync_copy(dst_hbm.at[wid], dst_v)

    lane = lax.iota(jnp.int32, 16)
    e0 = jnp.where(lane == 0, 1.0, 0.0).astype(jnp.float32)
    zeros = jnp.zeros((16,), jnp.float32)

    def fill(r, carry):
        msg_v[r] = e0
        tmp_v[r] = zeros
        return carry

    lax.fori_loop(0, EB, fill, 0)

    def zcopy(k, carry):
        pltpu.sync_copy(tmp_v, hist_shared.at[pl.ds(s * ROWS_PT + k * EB, EB)])
        return carry

    lax.fori_loop(0, RB, zcopy, 0)
    plsc.subcore_barrier()

    def body(b, carry):
        pltpu.sync_copy(msg_v, hist_shared.at[dst_v.at[b]], add=True)
        return carry

    lax.fori_loop(0, NB, body, 0)
    plsc.subcore_barrier()

    def dump(k, carry):
        r0 = s * ROWS_PT + k * EB
        pltpu.sync_copy(hist_shared.at[pl.ds(r0, EB)], tmp_v)
        pltpu.sync_copy(tmp_v, out_hbm.at[c].at[pl.ds(r0, EB)])
        return carry

    lax.fori_loop(0, RB, dump, 0)


def _prep_body(h0_ref, h1_ref, x_ref, y_ref, s2_ref):
    deg = jnp.sum(h0_ref[...] + h1_ref[...], axis=1, keepdims=True) + 1.0
    dinv = lax.rsqrt(deg)
    s2_ref[...] = dinv / deg
    y_ref[...] = x_ref[...] * dinv


@functools.partial(
    pl.kernel,
    out_type=jax.ShapeDtypeStruct((NC, N_PAD, D), jnp.float32),
    mesh=_mesh,
    scratch_types=[
        pltpu.VMEM((NB, EB), jnp.int32),
        pltpu.VMEM((NB, EB), jnp.int32),
        pltpu.VMEM((EB, D), jnp.float32),
        pltpu.VMEM_SHARED((N_PAD, D), jnp.float32),
        pltpu.SemaphoreType.DMA,
    ],
)
def _scatter_kernel(src_hbm, dst_hbm, y_hbm, out_hbm,
                    src_v, dst_v, rows_v, s_shared, sem):
    c = lax.axis_index("c")
    s = lax.axis_index("s")
    wid = s * NC + c
    pltpu.sync_copy(src_hbm.at[wid], src_v)
    pltpu.sync_copy(dst_hbm.at[wid], dst_v)

    zeros = jnp.zeros((16,), jnp.float32)

    def zrow(i, carry):
        r = i // (D // 16)
        col = (i % (D // 16)) * 16
        rows_v[r, pl.ds(col, 16)] = zeros
        return carry

    lax.fori_loop(0, EB * D // 16, zrow, 0)

    def zcopy(k, carry):
        pltpu.sync_copy(rows_v, s_shared.at[pl.ds(s * ROWS_PT + k * EB, EB)])
        return carry

    lax.fori_loop(0, RB, zcopy, 0)
    plsc.subcore_barrier()

    def body(b, carry):
        pltpu.async_copy(y_hbm.at[src_v.at[b]], rows_v, sem).wait()
        pltpu.sync_copy(rows_v, s_shared.at[dst_v.at[b]], add=True)
        return carry

    lax.fori_loop(0, NB, body, 0)
    plsc.subcore_barrier()

    def dump(k, carry):
        r0 = s * ROWS_PT + k * EB
        pltpu.sync_copy(s_shared.at[pl.ds(r0, EB)], rows_v)
        pltpu.sync_copy(rows_v, out_hbm.at[c].at[pl.ds(r0, EB)])
        return carry

    lax.fori_loop(0, RB, dump, 0)


def _final_body(s0_ref, s1_ref, y_ref, s2_ref, w1_ref, w2_ref, o_ref):
    a = (s0_ref[...] + s1_ref[...] + y_ref[...]) * s2_ref[...]
    h1 = jnp.dot(a, w1_ref[...], preferred_element_type=jnp.float32)
    h2 = jnp.dot(a, w2_ref[...], preferred_element_type=jnp.float32)
    o_ref[...] = jax.nn.relu(h1) * jax.nn.sigmoid(h2)


def kernel(x, edge_index, W1, W2):
    src = edge_index[0].astype(jnp.int32)
    dst = edge_index[1].astype(jnp.int32)
    pad = E_PAD - N_EDGES
    ar = jnp.arange(pad, dtype=jnp.int32)
    src_p = jnp.concatenate([src, ar % N_NODES]).reshape(NW, NB, EB)
    dst_pad = (ar % (N_PAD - N_NODES)) + N_NODES
    dst_p = jnp.concatenate([dst, dst_pad]).reshape(NW, NB, EB)

    hist = _deg_kernel(dst_p)

    x_p = jnp.concatenate(
        [x, jnp.zeros((N_PAD - N_NODES, D), jnp.float32)], axis=0)
    blk = 1280
    grid = N_PAD // blk
    y, s2 = pl.pallas_call(
        _prep_body,
        grid=(grid,),
        in_specs=[
            pl.BlockSpec((blk, HW), lambda i: (i, 0)),
            pl.BlockSpec((blk, HW), lambda i: (i, 0)),
            pl.BlockSpec((blk, D), lambda i: (i, 0)),
        ],
        out_specs=(
            pl.BlockSpec((blk, D), lambda i: (i, 0)),
            pl.BlockSpec((blk, 1), lambda i: (i, 0)),
        ),
        out_shape=(
            jax.ShapeDtypeStruct((N_PAD, D), jnp.float32),
            jax.ShapeDtypeStruct((N_PAD, 1), jnp.float32),
        ),
    )(hist[0], hist[1], x_p)

    S = _scatter_kernel(src_p, dst_p, y)

    out = pl.pallas_call(
        _final_body,
        grid=(grid,),
        in_specs=[
            pl.BlockSpec((blk, D), lambda i: (i, 0)),
            pl.BlockSpec((blk, D), lambda i: (i, 0)),
            pl.BlockSpec((blk, D), lambda i: (i, 0)),
            pl.BlockSpec((blk, 1), lambda i: (i, 0)),
            pl.BlockSpec((D, D), lambda i: (0, 0)),
            pl.BlockSpec((D, D), lambda i: (0, 0)),
        ],
        out_specs=pl.BlockSpec((blk, D), lambda i: (i, 0)),
        out_shape=jax.ShapeDtypeStruct((N_PAD, D), jnp.float32),
    )(S[0], S[1], y, s2, W1, W2)
    return out[:N_NODES]


# speedup vs baseline: 45.7548x; 1.3234x over previous
import functools

import jax
import jax.numpy as jnp
from jax import lax
from jax.experimental import pallas as pl
from jax.experimental.pallas import tpu as pltpu
from jax.experimental.pallas import tpu_sc as plsc

N_NODES = 10000
N_EDGES = 320000
D = 128

NC = 2
NS = 16
NW = NC * NS

N_PAD = 10240
E_PAD = 327680
EPT = E_PAD // NW
EB = 128
NB = EPT // EB
HNB = NB // 2
ROWS_PT = N_PAD // NS
RB = ROWS_PT // EB

_mesh = plsc.VectorSubcoreMesh(core_axis_name="c", subcore_axis_name="s")


HW = 16


@functools.partial(
    pl.kernel,
    out_type=jax.ShapeDtypeStruct((NC, N_PAD, HW), jnp.float32),
    mesh=_mesh,
    scratch_types=[
        pltpu.VMEM((NB, EB), jnp.int32),
        pltpu.VMEM((EB, HW), jnp.float32),
        pltpu.VMEM((EB, HW), jnp.float32),
        pltpu.VMEM_SHARED((N_PAD, HW), jnp.float32),
    ],
)
def _deg_kernel(dst_hbm, out_hbm, dst_v, msg_v, tmp_v, hist_shared):
    c = lax.axis_index("c")
    s = lax.axis_index("s")
    wid = s * NC + c
    pltpu.sync_copy(dst_hbm.at[wid], dst_v)

    lane = lax.iota(jnp.int32, 16)
    e0 = jnp.where(lane == 0, 1.0, 0.0).astype(jnp.float32)
    zeros = jnp.zeros((16,), jnp.float32)

    def fill(r, carry):
        msg_v[r] = e0
        tmp_v[r] = zeros
        return carry

    lax.fori_loop(0, EB, fill, 0)

    def zcopy(k, carry):
        pltpu.sync_copy(tmp_v, hist_shared.at[pl.ds(s * ROWS_PT + k * EB, EB)])
        return carry

    lax.fori_loop(0, RB, zcopy, 0)
    plsc.subcore_barrier()

    def body(b, carry):
        pltpu.sync_copy(msg_v, hist_shared.at[dst_v.at[b]], add=True)
        return carry

    lax.fori_loop(0, NB, body, 0)
    plsc.subcore_barrier()

    def dump(k, carry):
        r0 = s * ROWS_PT + k * EB
        pltpu.sync_copy(hist_shared.at[pl.ds(r0, EB)], tmp_v)
        pltpu.sync_copy(tmp_v, out_hbm.at[c].at[pl.ds(r0, EB)])
        return carry

    lax.fori_loop(0, RB, dump, 0)


def _prep_body(h0_ref, h1_ref, x_ref, y_ref, s2_ref):
    deg = jnp.sum(h0_ref[...] + h1_ref[...], axis=1, keepdims=True) + 1.0
    dinv = lax.rsqrt(deg)
    s2_ref[...] = dinv / deg
    y_ref[...] = x_ref[...] * dinv


@functools.partial(
    pl.kernel,
    out_type=jax.ShapeDtypeStruct((NC, N_PAD, D), jnp.float32),
    mesh=_mesh,
    scratch_types=[
        pltpu.VMEM((HNB, EB), jnp.int32),
        pltpu.VMEM((HNB, EB), jnp.int32),
        pltpu.VMEM((EB, D), jnp.float32),
        pltpu.VMEM((EB, D), jnp.float32),
        pltpu.VMEM_SHARED((N_PAD, D), jnp.float32),
        pltpu.SemaphoreType.DMA,
        pltpu.SemaphoreType.DMA,
    ],
)
def _scatter_kernel(src_hbm, dst_hbm, y_hbm, out_hbm,
                    src_v, dst_v, rows_a, rows_b, s_shared, sem_a, sem_b):
    c = lax.axis_index("c")
    s = lax.axis_index("s")
    wid = s * NC + c

    zeros = jnp.zeros((16,), jnp.float32)

    def zrow(i, carry):
        r = i // (D // 16)
        col = (i % (D // 16)) * 16
        rows_a[r, pl.ds(col, 16)] = zeros
        return carry

    lax.fori_loop(0, EB * D // 16, zrow, 0)

    def zcopy(k, carry):
        pltpu.sync_copy(rows_a, s_shared.at[pl.ds(s * ROWS_PT + k * EB, EB)])
        return carry

    lax.fori_loop(0, RB, zcopy, 0)
    plsc.subcore_barrier()

    for p in range(2):
        pltpu.sync_copy(src_hbm.at[wid].at[pl.ds(p * HNB, HNB)], src_v)
        pltpu.sync_copy(dst_hbm.at[wid].at[pl.ds(p * HNB, HNB)], dst_v)
        pltpu.async_copy(y_hbm.at[src_v.at[0]], rows_a, sem_a)

        def body(i, carry):
            b = 2 * i
            pltpu.async_copy(y_hbm.at[src_v.at[b + 1]], rows_b, sem_b)
            pltpu.make_async_copy(y_hbm.at[src_v.at[b]], rows_a, sem_a).wait()
            pltpu.sync_copy(rows_a, s_shared.at[dst_v.at[b]], add=True)

            @pl.when(b + 2 < HNB)
            def _():
                pltpu.async_copy(y_hbm.at[src_v.at[b + 2]], rows_a, sem_a)

            pltpu.make_async_copy(y_hbm.at[src_v.at[b + 1]], rows_b, sem_b).wait()
            pltpu.sync_copy(rows_b, s_shared.at[dst_v.at[b + 1]], add=True)
            return carry

        lax.fori_loop(0, HNB // 2, body, 0)
    plsc.subcore_barrier()

    def dump(k, carry):
        r0 = s * ROWS_PT + k * EB
        pltpu.sync_copy(s_shared.at[pl.ds(r0, EB)], rows_a)
        pltpu.sync_copy(rows_a, out_hbm.at[c].at[pl.ds(r0, EB)])
        return carry

    lax.fori_loop(0, RB, dump, 0)


def _final_body(s0_ref, s1_ref, y_ref, s2_ref, w1_ref, w2_ref, o_ref):
    a = (s0_ref[...] + s1_ref[...] + y_ref[...]) * s2_ref[...]
    h1 = jnp.dot(a, w1_ref[...], preferred_element_type=jnp.float32)
    h2 = jnp.dot(a, w2_ref[...], preferred_element_type=jnp.float32)
    o_ref[...] = jax.nn.relu(h1) * jax.nn.sigmoid(h2)


def kernel(x, edge_index, W1, W2):
    src = edge_index[0].astype(jnp.int32)
    dst = edge_index[1].astype(jnp.int32)
    pad = E_PAD - N_EDGES
    ar = jnp.arange(pad, dtype=jnp.int32)
    src_p = jnp.concatenate([src, ar % N_NODES]).reshape(NW, NB, EB)
    dst_pad = (ar % (N_PAD - N_NODES)) + N_NODES
    dst_p = jnp.concatenate([dst, dst_pad]).reshape(NW, NB, EB)

    hist = _deg_kernel(dst_p)

    x_p = jnp.concatenate(
        [x, jnp.zeros((N_PAD - N_NODES, D), jnp.float32)], axis=0)
    blk = 1280
    grid = N_PAD // blk
    y, s2 = pl.pallas_call(
        _prep_body,
        grid=(grid,),
        in_specs=[
            pl.BlockSpec((blk, HW), lambda i: (i, 0)),
            pl.BlockSpec((blk, HW), lambda i: (i, 0)),
            pl.BlockSpec((blk, D), lambda i: (i, 0)),
        ],
        out_specs=(
            pl.BlockSpec((blk, D), lambda i: (i, 0)),
            pl.BlockSpec((blk, 1), lambda i: (i, 0)),
        ),
        out_shape=(
            jax.ShapeDtypeStruct((N_PAD, D), jnp.float32),
            jax.ShapeDtypeStruct((N_PAD, 1), jnp.float32),
        ),
    )(hist[0], hist[1], x_p)

    S = _scatter_kernel(src_p, dst_p, y)

    out = pl.pallas_call(
        _final_body,
        grid=(grid,),
        in_specs=[
            pl.BlockSpec((blk, D), lambda i: (i, 0)),
            pl.BlockSpec((blk, D), lambda i: (i, 0)),
            pl.BlockSpec((blk, D), lambda i: (i, 0)),
            pl.BlockSpec((blk, 1), lambda i: (i, 0)),
            pl.BlockSpec((D, D), lambda i: (0, 0)),
            pl.BlockSpec((D, D), lambda i: (0, 0)),
        ],
        out_specs=pl.BlockSpec((blk, D), lambda i: (i, 0)),
        out_shape=jax.ShapeDtypeStruct((N_PAD, D), jnp.float32),
    )(S[0], S[1], y, s2, W1, W2)
    return out[:N_NODES]
